# baseline (speedup 1.0000x reference)
from functools import partial

import numpy as np

import jax
import jax.numpy as jnp
from jax.experimental import pallas as pl
from jax.experimental.pallas import tpu as pltpu
from jax.experimental.shard_map import shard_map
from jax.sharding import Mesh, PartitionSpec as P

HIDDEN = 128
IN_EXT = 5
LANE = 128


def _cdiv(a, b):
    return (a + b - 1) // b


def _fused_kernel(xa_ref, w1e_ref, w2t_ref, b2_ref, o_ref):
    h = jnp.dot(w1e_ref[...], xa_ref[...],
                preferred_element_type=jnp.float32)
    h = jnp.maximum(h.astype(jnp.bfloat16), jnp.bfloat16(0.0))
    q = jnp.dot(w2t_ref[...], h,
                preferred_element_type=jnp.float32)
    o_ref[...] = q + b2_ref[0, 0]


def _forward(x, a, w1, b1, w2, b2, tile_b):
    Bl = x.shape[0]
    nt = _cdiv(Bl, tile_b)
    if nt > 1 and nt % 2 == 1:
        nt += 1
    B_pad = nt * tile_b

    ones = jnp.ones((Bl, 1), x.dtype)
    xa = jnp.concatenate([x, a, ones], axis=-1)
    if B_pad != Bl:
        xa = jnp.pad(xa, ((0, B_pad - Bl), (0, 0)))
    xa_t = xa.T.astype(jnp.bfloat16)

    w1e = jnp.concatenate([w1, b1.reshape(1, HIDDEN)],
                          axis=0).T.astype(jnp.bfloat16)
    w2t = w2.reshape(1, HIDDEN).astype(jnp.bfloat16)
    b2s = b2.reshape(1, 1)

    q_t = pl.pallas_call(
        _fused_kernel,
        out_shape=jax.ShapeDtypeStruct((1, B_pad), jnp.float32),
        grid=(nt,),
        in_specs=[
            pl.BlockSpec((IN_EXT, tile_b), lambda i: (0, i)),
            pl.BlockSpec((HIDDEN, IN_EXT), lambda i: (0, 0)),
            pl.BlockSpec((1, HIDDEN), lambda i: (0, 0)),
            pl.BlockSpec((1, 1), lambda i: (0, 0),
                         memory_space=pltpu.SMEM),
        ],
        out_specs=pl.BlockSpec((1, tile_b), lambda i: (0, i)),
        compiler_params=pltpu.CompilerParams(
            dimension_semantics=("parallel",)),
    )(xa_t, w1e, w2t, b2s)

    return q_t.reshape(B_pad, 1)[:Bl]


def kernel(x, a, w1, b1, w2, b2):
    B = x.shape[0]
    TB = 65536

    devs = jax.devices()
    if len(devs) >= 2 and B % (2 * TB) == 0:
        mesh = Mesh(np.asarray(devs[:2]), ("b",))
        fn = shard_map(
            partial(_forward, tile_b=TB),
            mesh=mesh,
            in_specs=(P("b", None), P("b", None),
                      P(None, None), P(None, None),
                      P(None, None), P(None, None)),
            out_specs=P("b", None),
            check_rep=False,
        )
        return fn(x, a, w1, b1, w2, b2)

    return _forward(x, a, w1, b1, w2, b2, TB)


# speedup vs baseline: 2.3432x; 2.3432x over previous
import jax
import jax.numpy as jnp
from jax.experimental import pallas as pl
from jax.experimental.pallas import tpu as pltpu

HIDDEN = 128
IN_EXT = 5
LANE = 128


def _cdiv(a, b):
    return (a + b - 1) // b


def _fused_kernel(xa_ref, w1e_ref, w2t_ref, b2_ref, o_ref):
    h = jnp.dot(w1e_ref[...], xa_ref[...],
                preferred_element_type=jnp.float32)
    h = jnp.maximum(h.astype(jnp.bfloat16), jnp.bfloat16(0.0))
    q = jnp.dot(w2t_ref[...], h,
                preferred_element_type=jnp.float32)
    o_ref[...] = q + b2_ref[0, 0]


def kernel(x, a, w1, b1, w2, b2):
    B = x.shape[0]
    TB = 65536
    nt = _cdiv(B, TB)
    if nt > 1 and nt % 2 == 1:
        nt += 1
    B_pad = nt * TB

    ones = jnp.ones((B, 1), x.dtype)
    xa = jnp.concatenate([x, a, ones], axis=-1)
    if B_pad != B:
        xa = jnp.pad(xa, ((0, B_pad - B), (0, 0)))
    xa_t = xa.T.astype(jnp.bfloat16)

    w1e = jnp.concatenate([w1, b1.reshape(1, HIDDEN)],
                          axis=0).T.astype(jnp.bfloat16)
    w2t = w2.reshape(1, HIDDEN).astype(jnp.bfloat16)
    b2s = b2.reshape(1, 1)

    q_t = pl.pallas_call(
        _fused_kernel,
        out_shape=jax.ShapeDtypeStruct((1, B_pad), jnp.float32),
        grid=(nt,),
        in_specs=[
            pl.BlockSpec((IN_EXT, TB), lambda i: (0, i)),
            pl.BlockSpec((HIDDEN, IN_EXT), lambda i: (0, 0)),
            pl.BlockSpec((1, HIDDEN), lambda i: (0, 0)),
            pl.BlockSpec((1, 1), lambda i: (0, 0),
                         memory_space=pltpu.SMEM),
        ],
        out_specs=pl.BlockSpec((1, TB), lambda i: (0, i)),
        compiler_params=pltpu.CompilerParams(
            dimension_semantics=("parallel",)),
    )(xa_t, w1e, w2t, b2s)

    return q_t.reshape(B_pad, 1)[:B]
